# baseline (speedup 1.0000x reference)
import functools

import jax
import jax.numpy as jnp
from jax import lax
from jax.experimental import pallas as pl
from jax.experimental.pallas import tpu as pltpu
from jax.experimental.pallas import tpu_sc as plsc

D = 100
DG = 128
HIDDEN = 250
HP = 256
G = 4 * HP
B = 64
P = 16
T = 16
N = B * P
NUM_REL = 12
SMALL = 96
SDIM = 16
NIDX = N * T + 2 * B
NPAD_H = 8448
CHUNK = 24
TS_A = (0, 1, 2, 3, 12, 13, 14, 15)
TS_B = (4, 5, 6, 7, 8, 9, 10, 11)


def _pad_cast_body(src_ref, dst_ref):
    x = src_ref[...]
    xt = jnp.transpose(x, (1, 0))
    z = jnp.zeros((xt.shape[0], DG - D), xt.dtype)
    dst_ref[...] = jnp.concatenate([xt, z], axis=1)


def _pad_cast(name_emb_t):
    v, rows = name_emb_t.shape[1], 16384
    return pl.pallas_call(
        _pad_cast_body,
        grid=(pl.cdiv(v, rows),),
        in_specs=[pl.BlockSpec((D, rows), lambda i: (0, i))],
        out_specs=pl.BlockSpec((rows, DG), lambda i: (i, 0)),
        out_shape=jax.ShapeDtypeStruct((v, DG), jnp.float32),
    )(name_emb_t)


def _sc_gather(name_emb, idx_all):
    info = plsc.get_sparse_core_info()
    nc, ns = info.num_cores, info.num_subcores
    nw = nc * ns
    bpw = NPAD_H // nw
    nchunk = bpw // CHUNK
    mesh = plsc.VectorSubcoreMesh(core_axis_name="c", subcore_axis_name="s")

    @functools.partial(
        pl.kernel,
        mesh=mesh,
        out_type=jax.ShapeDtypeStruct((NPAD_H, DG), jnp.float32),
        scratch_types=[
            pltpu.VMEM((nchunk, CHUNK), jnp.int32),
            pltpu.VMEM((bpw, DG), jnp.float32),
            pltpu.SemaphoreType.DMA,
        ],
    )
    def k(table_hbm, idx_hbm, out_hbm, idx_v, rows_v, sem):
        wid = lax.axis_index("s") * nc + lax.axis_index("c")
        base = wid * bpw
        pltpu.sync_copy(idx_hbm.at[wid], idx_v)
        copies = []
        for j in range(nchunk):
            copies.append(
                pltpu.async_copy(
                    table_hbm.at[idx_v.at[j]],
                    rows_v.at[pl.ds(j * CHUNK, CHUNK)],
                    sem,
                )
            )
        for c in copies:
            c.wait()
        pltpu.sync_copy(rows_v, out_hbm.at[pl.ds(base, bpw)])

    return k(name_emb, idx_all.reshape(nw, nchunk, CHUNK))


def _lstm_helpers(idxs_ref, sblk_ref, wsm_f_ref, wsm_b_ref):
    f32 = jnp.float32
    bf16 = jnp.bfloat16
    dot = functools.partial(jnp.dot, preferred_element_type=f32)
    sg_f = dot(sblk_ref[...], wsm_f_ref[...]).astype(bf16)
    sg_b = dot(sblk_ref[...], wsm_b_ref[...]).astype(bf16)
    iota = lax.broadcasted_iota(jnp.int32, (N, SMALL), 1)

    def onehot(t):
        p_ = idxs_ref[0, :, t:t + 1]
        d_ = idxs_ref[1, :, t:t + 1]
        r_ = idxs_ref[2, :, t:t + 1]
        hit = (iota == p_) | (iota == d_) | (iota == r_)
        return hit.astype(bf16)

    def cell(gates, h, c, upd):
        ig = jax.nn.sigmoid(gates[:, 0:HP])
        fg = jax.nn.sigmoid(gates[:, HP:2 * HP])
        gg = jnp.tanh(gates[:, 2 * HP:3 * HP])
        og = jax.nn.sigmoid(gates[:, 3 * HP:4 * HP])
        c2 = fg * c + ig * gg
        h2 = og * jnp.tanh(c2)
        return jnp.where(upd, h2, h), jnp.where(upd, c2, c)

    return sg_f, sg_b, onehot, cell


def _word_at(rowsA_ref, rowsB_ref, idxs_ref, t):
    if t in TS_B:
        x = rowsB_ref[pl.ds(TS_B.index(t) * N, N), :]
    else:
        x = rowsA_ref[pl.ds(TS_A.index(t) * N, N), :]
    return x.astype(jnp.bfloat16)


def _tc_body1(rowsA_ref, idxs_ref, len_ref, sblk_ref, wsm_f_ref, wsm_b_ref,
              ww_f_ref, ww_b_ref, whh_f_ref, whh_b_ref, bias_f_ref,
              bias_b_ref, hc_ref):
    f32 = jnp.float32
    bf16 = jnp.bfloat16
    dot = functools.partial(jnp.dot, preferred_element_type=f32)
    sg_f, sg_b, onehot, cell = _lstm_helpers(
        idxs_ref, sblk_ref, wsm_f_ref, wsm_b_ref)
    lens = len_ref[...]
    bias_f = bias_f_ref[...]
    bias_b = bias_b_ref[...]
    whh_f = whh_f_ref[...]
    whh_b = whh_b_ref[...]
    h_f = jnp.zeros((N, HP), f32)
    c_f = jnp.zeros((N, HP), f32)
    h_b = jnp.zeros((N, HP), f32)
    c_b = jnp.zeros((N, HP), f32)
    for s in range(4):
        tb = T - 1 - s
        g_f = (dot(_word_at(rowsA_ref, None, idxs_ref, s), ww_f_ref[...])
               + dot(onehot(s), sg_f) + dot(h_f.astype(bf16), whh_f) + bias_f)
        h_f, c_f = cell(g_f, h_f, c_f, lens > s)
        g_b = (dot(_word_at(rowsA_ref, None, idxs_ref, tb), ww_b_ref[...])
               + dot(onehot(tb), sg_b) + dot(h_b.astype(bf16), whh_b) + bias_b)
        h_b, c_b = cell(g_b, h_b, c_b, lens > tb)
    hc_ref[0] = h_f
    hc_ref[1] = c_f
    hc_ref[2] = h_b
    hc_ref[3] = c_b


def _tc_body2(rowsA_ref, rowsB_ref, idxs_ref, len_ref, counts_ref,
              hc_ref, sblk_ref, wsm_f_ref, wsm_b_ref, ww_f_ref, ww_b_ref,
              whh_f_ref, whh_b_ref, bias_f_ref, bias_b_ref,
              wout_n_ref, wout_f_ref, wout_b_ref, bout_ref, out_ref):
    f32 = jnp.float32
    bf16 = jnp.bfloat16
    dot = functools.partial(jnp.dot, preferred_element_type=f32)
    sg_f, sg_b, onehot, cell = _lstm_helpers(
        idxs_ref, sblk_ref, wsm_f_ref, wsm_b_ref)
    lens = len_ref[...]
    bias_f = bias_f_ref[...]
    bias_b = bias_b_ref[...]
    whh_f = whh_f_ref[...]
    whh_b = whh_b_ref[...]
    h_f = hc_ref[0]
    c_f = hc_ref[1]
    h_b = hc_ref[2]
    c_b = hc_ref[3]
    for s in range(4, T):
        tb = T - 1 - s
        g_f = (dot(_word_at(rowsA_ref, rowsB_ref, idxs_ref, s), ww_f_ref[...])
               + dot(onehot(s), sg_f) + dot(h_f.astype(bf16), whh_f) + bias_f)
        h_f, c_f = cell(g_f, h_f, c_f, lens > s)
        g_b = (dot(_word_at(rowsA_ref, rowsB_ref, idxs_ref, tb), ww_b_ref[...])
               + dot(onehot(tb), sg_b) + dot(h_b.astype(bf16), whh_b) + bias_b)
        h_b, c_b = cell(g_b, h_b, c_b, lens > tb)

    counts3 = counts_ref[...]
    pw_f = jnp.sum(h_f.reshape(B, P, HP) * counts3, axis=1)
    pw_b = jnp.sum(h_b.reshape(B, P, HP) * counts3, axis=1)
    nodes = rowsA_ref[pl.ds(8 * N, 2 * B), :].reshape(B, 2 * DG)
    logits = (dot(nodes, wout_n_ref[...]) + dot(pw_f, wout_f_ref[...])
              + dot(pw_b, wout_b_ref[...]) + bout_ref[...])
    mx = jnp.max(logits, axis=-1, keepdims=True)
    s_ = logits - mx
    lse = jnp.log(jnp.sum(jnp.exp(s_), axis=-1, keepdims=True))
    out_ref[...] = s_ - lse


def _pad_gate_rows(w):
    w4 = w.reshape(4, HIDDEN, -1)
    w4 = jnp.pad(w4, ((0, 0), (0, HP - HIDDEN), (0, 0)))
    return w4.reshape(G, -1)


def kernel(nodes, paths, counts, edgecounts, max_paths, max_edges, name_emb,
           pos_emb, dep_emb, dir_emb, W_ih_f, W_hh_f, b_ih_f, b_hh_f,
           W_ih_b, W_hh_b, b_ih_b, b_hh_b, W_out, b_out):
    i32 = jnp.int32
    word_idx = paths[..., 0].reshape(N, T).T.astype(i32)
    node_idx = nodes.reshape(-1).astype(i32)
    idx_a = jnp.concatenate(
        [word_idx[jnp.array(TS_A)].reshape(-1), node_idx,
         jnp.zeros((NPAD_H - 8 * N - 2 * B,), i32)])
    idx_b = jnp.concatenate(
        [word_idx[jnp.array(TS_B)].reshape(-1),
         jnp.zeros((NPAD_H - 8 * N,), i32)])

    table = _pad_cast(name_emb.T)
    rows_a = _sc_gather(table, idx_a)
    rows_b = _sc_gather(table, idx_b)

    pos_i = paths[..., 1].reshape(N, T).astype(i32)
    dep_i = paths[..., 2].reshape(N, T).astype(i32) + 40
    dir_i = paths[..., 3].reshape(N, T).astype(i32) + 90
    idxs = jnp.stack([pos_i, dep_i, dir_i])
    lens = edgecounts.reshape(N, 1).astype(i32)
    counts3 = counts.astype(jnp.float32).reshape(B, P, 1)

    sblk = jnp.zeros((SMALL, SDIM), jnp.float32)
    sblk = sblk.at[0:40, 0:4].set(pos_emb)
    sblk = sblk.at[40:90, 4:10].set(dep_emb)
    sblk = sblk.at[90:94, 10:13].set(dir_emb)

    def split_ih(w_ih):
        wp = _pad_gate_rows(w_ih)
        ww = jnp.pad(wp[:, :D].T, ((0, DG - D), (0, 0)))
        wsm = jnp.pad(wp[:, D:].T, ((0, SDIM - 13), (0, 0)))
        return ww.astype(jnp.bfloat16), wsm

    ww_f, wsm_f = split_ih(W_ih_f)
    ww_b, wsm_b = split_ih(W_ih_b)
    whh_f = jnp.pad(_pad_gate_rows(W_hh_f),
                    ((0, 0), (0, HP - HIDDEN))).T.astype(jnp.bfloat16)
    whh_b = jnp.pad(_pad_gate_rows(W_hh_b),
                    ((0, 0), (0, HP - HIDDEN))).T.astype(jnp.bfloat16)
    bias_f = _pad_gate_rows((b_ih_f + b_hh_f)[:, None]).reshape(1, G)
    bias_b = _pad_gate_rows((b_ih_b + b_hh_b)[:, None]).reshape(1, G)
    wout_n = jnp.zeros((2 * DG, NUM_REL), jnp.float32)
    wout_n = wout_n.at[0:D].set(W_out[:, :D].T)
    wout_n = wout_n.at[DG:DG + D].set(W_out[:, D:2 * D].T)
    wout_f = jnp.pad(W_out[:, 2 * D::2].T, ((0, HP - HIDDEN), (0, 0)))
    wout_b = jnp.pad(W_out[:, 2 * D + 1::2].T, ((0, HP - HIDDEN), (0, 0)))
    bout = b_out.reshape(1, NUM_REL)

    hc = pl.pallas_call(
        _tc_body1,
        out_shape=jax.ShapeDtypeStruct((4, N, HP), jnp.float32),
    )(rows_a, idxs, lens, sblk, wsm_f, wsm_b, ww_f, ww_b,
      whh_f, whh_b, bias_f, bias_b)
    out = pl.pallas_call(
        _tc_body2,
        out_shape=jax.ShapeDtypeStruct((B, NUM_REL), jnp.float32),
    )(rows_a, rows_b, idxs, lens, counts3, hc, sblk, wsm_f, wsm_b,
      ww_f, ww_b, whh_f, whh_b, bias_f, bias_b, wout_n, wout_f, wout_b, bout)
    return out


# speedup vs baseline: 1.0034x; 1.0034x over previous
import functools

import jax
import jax.numpy as jnp
from jax import lax
from jax.experimental import pallas as pl
from jax.experimental.pallas import tpu as pltpu
from jax.experimental.pallas import tpu_sc as plsc

D = 100
DG = 128
HIDDEN = 250
HP = 256
G = 4 * HP
B = 64
P = 16
T = 16
N = B * P
NUM_REL = 12
SMALL = 96
SDIM = 16
NIDX = N * T + 2 * B
NPAD_H = 8448
CHUNK = 24
TS_A = (0, 1, 2, 3, 12, 13, 14, 15)
TS_B = (4, 5, 6, 7, 8, 9, 10, 11)


def _pad_cast_body(src_ref, dst_ref):
    x = src_ref[...]
    xt = jnp.transpose(x, (1, 0))
    z = jnp.zeros((xt.shape[0], DG - D), xt.dtype)
    dst_ref[...] = jnp.concatenate([xt, z], axis=1)


def _pad_cast(name_emb_t):
    v, rows = name_emb_t.shape[1], 32768
    return pl.pallas_call(
        _pad_cast_body,
        grid=(pl.cdiv(v, rows),),
        in_specs=[pl.BlockSpec((D, rows), lambda i: (0, i))],
        out_specs=pl.BlockSpec((rows, DG), lambda i: (i, 0)),
        out_shape=jax.ShapeDtypeStruct((v, DG), jnp.float32),
    )(name_emb_t)


def _sc_gather(name_emb, idx_all):
    info = plsc.get_sparse_core_info()
    nc, ns = info.num_cores, info.num_subcores
    nw = nc * ns
    bpw = NPAD_H // nw
    nchunk = bpw // CHUNK
    mesh = plsc.VectorSubcoreMesh(core_axis_name="c", subcore_axis_name="s")

    @functools.partial(
        pl.kernel,
        mesh=mesh,
        out_type=jax.ShapeDtypeStruct((NPAD_H, DG), jnp.float32),
        scratch_types=[
            pltpu.VMEM((nchunk, CHUNK), jnp.int32),
            pltpu.VMEM((bpw, DG), jnp.float32),
            pltpu.SemaphoreType.DMA,
        ],
    )
    def k(table_hbm, idx_hbm, out_hbm, idx_v, rows_v, sem):
        wid = lax.axis_index("s") * nc + lax.axis_index("c")
        base = wid * bpw
        pltpu.sync_copy(idx_hbm.at[wid], idx_v)
        copies = []
        for j in range(nchunk):
            copies.append(
                pltpu.async_copy(
                    table_hbm.at[idx_v.at[j]],
                    rows_v.at[pl.ds(j * CHUNK, CHUNK)],
                    sem,
                )
            )
        for c in copies:
            c.wait()
        pltpu.sync_copy(rows_v, out_hbm.at[pl.ds(base, bpw)])

    return k(name_emb, idx_all.reshape(nw, nchunk, CHUNK))


def _lstm_helpers(idxs_ref, sblk_ref, wsm_f_ref, wsm_b_ref):
    f32 = jnp.float32
    bf16 = jnp.bfloat16
    dot = functools.partial(jnp.dot, preferred_element_type=f32)
    sg_f = dot(sblk_ref[...], wsm_f_ref[...]).astype(bf16)
    sg_b = dot(sblk_ref[...], wsm_b_ref[...]).astype(bf16)
    iota = lax.broadcasted_iota(jnp.int32, (N, SMALL), 1)

    def onehot(t):
        p_ = idxs_ref[0, :, t:t + 1]
        d_ = idxs_ref[1, :, t:t + 1]
        r_ = idxs_ref[2, :, t:t + 1]
        hit = (iota == p_) | (iota == d_) | (iota == r_)
        return hit.astype(bf16)

    def cell(gates, h, c, upd):
        ig = jax.nn.sigmoid(gates[:, 0:HP])
        fg = jax.nn.sigmoid(gates[:, HP:2 * HP])
        gg = jnp.tanh(gates[:, 2 * HP:3 * HP])
        og = jax.nn.sigmoid(gates[:, 3 * HP:4 * HP])
        c2 = fg * c + ig * gg
        h2 = og * jnp.tanh(c2)
        return jnp.where(upd, h2, h), jnp.where(upd, c2, c)

    return sg_f, sg_b, onehot, cell


def _word_at(rowsA_ref, rowsB_ref, idxs_ref, t):
    if t in TS_B:
        x = rowsB_ref[pl.ds(TS_B.index(t) * N, N), :]
    else:
        x = rowsA_ref[pl.ds(TS_A.index(t) * N, N), :]
    return x.astype(jnp.bfloat16)


def _tc_body1(rowsA_ref, idxs_ref, len_ref, sblk_ref, wsm_f_ref, wsm_b_ref,
              ww_f_ref, ww_b_ref, whh_f_ref, whh_b_ref, bias_f_ref,
              bias_b_ref, hc_ref):
    f32 = jnp.float32
    bf16 = jnp.bfloat16
    dot = functools.partial(jnp.dot, preferred_element_type=f32)
    sg_f, sg_b, onehot, cell = _lstm_helpers(
        idxs_ref, sblk_ref, wsm_f_ref, wsm_b_ref)
    lens = len_ref[...]
    bias_f = bias_f_ref[...]
    bias_b = bias_b_ref[...]
    whh_f = whh_f_ref[...]
    whh_b = whh_b_ref[...]
    h_f = jnp.zeros((N, HP), f32)
    c_f = jnp.zeros((N, HP), f32)
    h_b = jnp.zeros((N, HP), f32)
    c_b = jnp.zeros((N, HP), f32)
    for s in range(4):
        tb = T - 1 - s
        g_f = (dot(_word_at(rowsA_ref, None, idxs_ref, s), ww_f_ref[...])
               + dot(onehot(s), sg_f) + dot(h_f.astype(bf16), whh_f) + bias_f)
        h_f, c_f = cell(g_f, h_f, c_f, lens > s)
        g_b = (dot(_word_at(rowsA_ref, None, idxs_ref, tb), ww_b_ref[...])
               + dot(onehot(tb), sg_b) + dot(h_b.astype(bf16), whh_b) + bias_b)
        h_b, c_b = cell(g_b, h_b, c_b, lens > tb)
    hc_ref[0] = h_f
    hc_ref[1] = c_f
    hc_ref[2] = h_b
    hc_ref[3] = c_b


def _tc_body2(rowsA_ref, rowsB_ref, idxs_ref, len_ref, counts_ref,
              hc_ref, sblk_ref, wsm_f_ref, wsm_b_ref, ww_f_ref, ww_b_ref,
              whh_f_ref, whh_b_ref, bias_f_ref, bias_b_ref,
              wout_n_ref, wout_f_ref, wout_b_ref, bout_ref, out_ref):
    f32 = jnp.float32
    bf16 = jnp.bfloat16
    dot = functools.partial(jnp.dot, preferred_element_type=f32)
    sg_f, sg_b, onehot, cell = _lstm_helpers(
        idxs_ref, sblk_ref, wsm_f_ref, wsm_b_ref)
    lens = len_ref[...]
    bias_f = bias_f_ref[...]
    bias_b = bias_b_ref[...]
    whh_f = whh_f_ref[...]
    whh_b = whh_b_ref[...]
    h_f = hc_ref[0]
    c_f = hc_ref[1]
    h_b = hc_ref[2]
    c_b = hc_ref[3]
    for s in range(4, T):
        tb = T - 1 - s
        g_f = (dot(_word_at(rowsA_ref, rowsB_ref, idxs_ref, s), ww_f_ref[...])
               + dot(onehot(s), sg_f) + dot(h_f.astype(bf16), whh_f) + bias_f)
        h_f, c_f = cell(g_f, h_f, c_f, lens > s)
        g_b = (dot(_word_at(rowsA_ref, rowsB_ref, idxs_ref, tb), ww_b_ref[...])
               + dot(onehot(tb), sg_b) + dot(h_b.astype(bf16), whh_b) + bias_b)
        h_b, c_b = cell(g_b, h_b, c_b, lens > tb)

    counts3 = counts_ref[...]
    pw_f = jnp.sum(h_f.reshape(B, P, HP) * counts3, axis=1)
    pw_b = jnp.sum(h_b.reshape(B, P, HP) * counts3, axis=1)
    nodes = rowsA_ref[pl.ds(8 * N, 2 * B), :].reshape(B, 2 * DG)
    logits = (dot(nodes, wout_n_ref[...]) + dot(pw_f, wout_f_ref[...])
              + dot(pw_b, wout_b_ref[...]) + bout_ref[...])
    mx = jnp.max(logits, axis=-1, keepdims=True)
    s_ = logits - mx
    lse = jnp.log(jnp.sum(jnp.exp(s_), axis=-1, keepdims=True))
    out_ref[...] = s_ - lse


def _pad_gate_rows(w):
    w4 = w.reshape(4, HIDDEN, -1)
    w4 = jnp.pad(w4, ((0, 0), (0, HP - HIDDEN), (0, 0)))
    return w4.reshape(G, -1)


def kernel(nodes, paths, counts, edgecounts, max_paths, max_edges, name_emb,
           pos_emb, dep_emb, dir_emb, W_ih_f, W_hh_f, b_ih_f, b_hh_f,
           W_ih_b, W_hh_b, b_ih_b, b_hh_b, W_out, b_out):
    i32 = jnp.int32
    word_idx = paths[..., 0].reshape(N, T).T.astype(i32)
    node_idx = nodes.reshape(-1).astype(i32)
    idx_a = jnp.concatenate(
        [word_idx[jnp.array(TS_A)].reshape(-1), node_idx,
         jnp.zeros((NPAD_H - 8 * N - 2 * B,), i32)])
    idx_b = jnp.concatenate(
        [word_idx[jnp.array(TS_B)].reshape(-1),
         jnp.zeros((NPAD_H - 8 * N,), i32)])

    table = _pad_cast(name_emb.T)
    rows_a = _sc_gather(table, idx_a)
    rows_b = _sc_gather(table, idx_b)

    pos_i = paths[..., 1].reshape(N, T).astype(i32)
    dep_i = paths[..., 2].reshape(N, T).astype(i32) + 40
    dir_i = paths[..., 3].reshape(N, T).astype(i32) + 90
    idxs = jnp.stack([pos_i, dep_i, dir_i])
    lens = edgecounts.reshape(N, 1).astype(i32)
    counts3 = counts.astype(jnp.float32).reshape(B, P, 1)

    sblk = jnp.zeros((SMALL, SDIM), jnp.float32)
    sblk = sblk.at[0:40, 0:4].set(pos_emb)
    sblk = sblk.at[40:90, 4:10].set(dep_emb)
    sblk = sblk.at[90:94, 10:13].set(dir_emb)

    def split_ih(w_ih):
        wp = _pad_gate_rows(w_ih)
        ww = jnp.pad(wp[:, :D].T, ((0, DG - D), (0, 0)))
        wsm = jnp.pad(wp[:, D:].T, ((0, SDIM - 13), (0, 0)))
        return ww.astype(jnp.bfloat16), wsm

    ww_f, wsm_f = split_ih(W_ih_f)
    ww_b, wsm_b = split_ih(W_ih_b)
    whh_f = jnp.pad(_pad_gate_rows(W_hh_f),
                    ((0, 0), (0, HP - HIDDEN))).T.astype(jnp.bfloat16)
    whh_b = jnp.pad(_pad_gate_rows(W_hh_b),
                    ((0, 0), (0, HP - HIDDEN))).T.astype(jnp.bfloat16)
    bias_f = _pad_gate_rows((b_ih_f + b_hh_f)[:, None]).reshape(1, G)
    bias_b = _pad_gate_rows((b_ih_b + b_hh_b)[:, None]).reshape(1, G)
    wout_n = jnp.zeros((2 * DG, NUM_REL), jnp.float32)
    wout_n = wout_n.at[0:D].set(W_out[:, :D].T)
    wout_n = wout_n.at[DG:DG + D].set(W_out[:, D:2 * D].T)
    wout_f = jnp.pad(W_out[:, 2 * D::2].T, ((0, HP - HIDDEN), (0, 0)))
    wout_b = jnp.pad(W_out[:, 2 * D + 1::2].T, ((0, HP - HIDDEN), (0, 0)))
    bout = b_out.reshape(1, NUM_REL)

    hc = pl.pallas_call(
        _tc_body1,
        out_shape=jax.ShapeDtypeStruct((4, N, HP), jnp.float32),
    )(rows_a, idxs, lens, sblk, wsm_f, wsm_b, ww_f, ww_b,
      whh_f, whh_b, bias_f, bias_b)
    out = pl.pallas_call(
        _tc_body2,
        out_shape=jax.ShapeDtypeStruct((B, NUM_REL), jnp.float32),
    )(rows_a, rows_b, idxs, lens, counts3, hc, sblk, wsm_f, wsm_b,
      ww_f, ww_b, whh_f, whh_b, bias_f, bias_b, wout_n, wout_f, wout_b, bout)
    return out


# speedup vs baseline: 1.0402x; 1.0366x over previous
import functools

import jax
import jax.numpy as jnp
from jax import lax
from jax.experimental import pallas as pl
from jax.experimental.pallas import tpu as pltpu
from jax.experimental.pallas import tpu_sc as plsc

D = 100
DG = 128
HIDDEN = 250
HP = 256
G = 4 * HP
B = 64
P = 16
T = 16
N = B * P
NUM_REL = 12
SMALL = 96
SDIM = 16
NIDX = N * T + 2 * B
NPAD_H = 8448
CHUNK = 24
TS_A = (0, 1, 2, 3, 12, 13, 14, 15)
TS_B = (4, 5, 6, 7, 8, 9, 10, 11)


def _pad_cast_body(src_ref, dst_ref):
    x = src_ref[...]
    xt = jnp.transpose(x, (1, 0))
    z = jnp.zeros((xt.shape[0], DG - D), xt.dtype)
    dst_ref[...] = jnp.concatenate([xt, z], axis=1)


def _pad_cast(name_emb_t):
    v, rows = name_emb_t.shape[1], 32768
    return pl.pallas_call(
        _pad_cast_body,
        grid=(pl.cdiv(v, rows),),
        in_specs=[pl.BlockSpec((D, rows), lambda i: (0, i))],
        out_specs=pl.BlockSpec((rows, DG), lambda i: (i, 0)),
        out_shape=jax.ShapeDtypeStruct((v, DG), jnp.float32),
    )(name_emb_t)


def _sc_gather(name_emb, idx_all, npad, chunk):
    info = plsc.get_sparse_core_info()
    nc, ns = info.num_cores, info.num_subcores
    nw = nc * ns
    bpw = npad // nw
    nchunk = bpw // chunk
    mesh = plsc.VectorSubcoreMesh(core_axis_name="c", subcore_axis_name="s")

    @functools.partial(
        pl.kernel,
        mesh=mesh,
        out_type=jax.ShapeDtypeStruct((npad, DG), jnp.float32),
        scratch_types=[
            pltpu.VMEM((nchunk, chunk), jnp.int32),
            pltpu.VMEM((bpw, DG), jnp.float32),
            pltpu.SemaphoreType.DMA,
        ],
    )
    def k(table_hbm, idx_hbm, out_hbm, idx_v, rows_v, sem):
        wid = lax.axis_index("s") * nc + lax.axis_index("c")
        base = wid * bpw
        pltpu.sync_copy(idx_hbm.at[wid], idx_v)
        copies = []
        for j in range(nchunk):
            copies.append(
                pltpu.async_copy(
                    table_hbm.at[idx_v.at[j]],
                    rows_v.at[pl.ds(j * chunk, chunk)],
                    sem,
                )
            )
        for c in copies:
            c.wait()
        pltpu.sync_copy(rows_v, out_hbm.at[pl.ds(base, bpw)])

    return k(name_emb, idx_all.reshape(nw, nchunk, chunk))


def _lstm_helpers(idxs_ref, sblk_ref, wsm_f_ref, wsm_b_ref):
    f32 = jnp.float32
    bf16 = jnp.bfloat16
    dot = functools.partial(jnp.dot, preferred_element_type=f32)
    sg_f = dot(sblk_ref[...], wsm_f_ref[...]).astype(bf16)
    sg_b = dot(sblk_ref[...], wsm_b_ref[...]).astype(bf16)
    iota = lax.broadcasted_iota(jnp.int32, (N, SMALL), 1)

    def onehot(t):
        p_ = idxs_ref[0, :, t:t + 1]
        d_ = idxs_ref[1, :, t:t + 1]
        r_ = idxs_ref[2, :, t:t + 1]
        hit = (iota == p_) | (iota == d_) | (iota == r_)
        return hit.astype(bf16)

    def cell(gates, h, c, upd):
        ig = jax.nn.sigmoid(gates[:, 0:HP])
        fg = jax.nn.sigmoid(gates[:, HP:2 * HP])
        gg = jnp.tanh(gates[:, 2 * HP:3 * HP])
        og = jax.nn.sigmoid(gates[:, 3 * HP:4 * HP])
        c2 = fg * c + ig * gg
        h2 = og * jnp.tanh(c2)
        return jnp.where(upd, h2, h), jnp.where(upd, c2, c)

    return sg_f, sg_b, onehot, cell


def _word_at(rowsA_ref, rowsB_ref, idxs_ref, t):
    if t in TS_B:
        x = rowsB_ref[pl.ds(TS_B.index(t) * N, N), :]
    else:
        x = rowsA_ref[pl.ds(TS_A.index(t) * N, N), :]
    return x.astype(jnp.bfloat16)


def _tc_body1(rowsA_ref, idxs_ref, len_ref, sblk_ref, wsm_f_ref, wsm_b_ref,
              ww_f_ref, ww_b_ref, whh_f_ref, whh_b_ref, bias_f_ref,
              bias_b_ref, hc_ref):
    f32 = jnp.float32
    bf16 = jnp.bfloat16
    dot = functools.partial(jnp.dot, preferred_element_type=f32)
    sg_f, sg_b, onehot, cell = _lstm_helpers(
        idxs_ref, sblk_ref, wsm_f_ref, wsm_b_ref)
    lens = len_ref[...]
    bias_f = bias_f_ref[...]
    bias_b = bias_b_ref[...]
    whh_f = whh_f_ref[...]
    whh_b = whh_b_ref[...]
    h_f = jnp.zeros((N, HP), f32)
    c_f = jnp.zeros((N, HP), f32)
    h_b = jnp.zeros((N, HP), f32)
    c_b = jnp.zeros((N, HP), f32)
    for s in range(4):
        tb = T - 1 - s
        g_f = (dot(_word_at(rowsA_ref, None, idxs_ref, s), ww_f_ref[...])
               + dot(onehot(s), sg_f) + dot(h_f.astype(bf16), whh_f) + bias_f)
        h_f, c_f = cell(g_f, h_f, c_f, lens > s)
        g_b = (dot(_word_at(rowsA_ref, None, idxs_ref, tb), ww_b_ref[...])
               + dot(onehot(tb), sg_b) + dot(h_b.astype(bf16), whh_b) + bias_b)
        h_b, c_b = cell(g_b, h_b, c_b, lens > tb)
    hc_ref[0] = h_f
    hc_ref[1] = c_f
    hc_ref[2] = h_b
    hc_ref[3] = c_b


def _tc_body2(rowsA_ref, rowsB_ref, idxs_ref, len_ref, counts_ref,
              hc_ref, sblk_ref, wsm_f_ref, wsm_b_ref, ww_f_ref, ww_b_ref,
              whh_f_ref, whh_b_ref, bias_f_ref, bias_b_ref,
              wout_n_ref, wout_f_ref, wout_b_ref, bout_ref, out_ref):
    f32 = jnp.float32
    bf16 = jnp.bfloat16
    dot = functools.partial(jnp.dot, preferred_element_type=f32)
    sg_f, sg_b, onehot, cell = _lstm_helpers(
        idxs_ref, sblk_ref, wsm_f_ref, wsm_b_ref)
    lens = len_ref[...]
    bias_f = bias_f_ref[...]
    bias_b = bias_b_ref[...]
    whh_f = whh_f_ref[...]
    whh_b = whh_b_ref[...]
    h_f = hc_ref[0]
    c_f = hc_ref[1]
    h_b = hc_ref[2]
    c_b = hc_ref[3]
    for s in range(4, T):
        tb = T - 1 - s
        g_f = (dot(_word_at(rowsA_ref, rowsB_ref, idxs_ref, s), ww_f_ref[...])
               + dot(onehot(s), sg_f) + dot(h_f.astype(bf16), whh_f) + bias_f)
        h_f, c_f = cell(g_f, h_f, c_f, lens > s)
        g_b = (dot(_word_at(rowsA_ref, rowsB_ref, idxs_ref, tb), ww_b_ref[...])
               + dot(onehot(tb), sg_b) + dot(h_b.astype(bf16), whh_b) + bias_b)
        h_b, c_b = cell(g_b, h_b, c_b, lens > tb)

    counts3 = counts_ref[...]
    pw_f = jnp.sum(h_f.reshape(B, P, HP) * counts3, axis=1)
    pw_b = jnp.sum(h_b.reshape(B, P, HP) * counts3, axis=1)
    nodes = rowsA_ref[pl.ds(8 * N, 2 * B), :].reshape(B, 2 * DG)
    logits = (dot(nodes, wout_n_ref[...]) + dot(pw_f, wout_f_ref[...])
              + dot(pw_b, wout_b_ref[...]) + bout_ref[...])
    mx = jnp.max(logits, axis=-1, keepdims=True)
    s_ = logits - mx
    lse = jnp.log(jnp.sum(jnp.exp(s_), axis=-1, keepdims=True))
    out_ref[...] = s_ - lse


def _pad_gate_rows(w):
    w4 = w.reshape(4, HIDDEN, -1)
    w4 = jnp.pad(w4, ((0, 0), (0, HP - HIDDEN), (0, 0)))
    return w4.reshape(G, -1)


def kernel(nodes, paths, counts, edgecounts, max_paths, max_edges, name_emb,
           pos_emb, dep_emb, dir_emb, W_ih_f, W_hh_f, b_ih_f, b_hh_f,
           W_ih_b, W_hh_b, b_ih_b, b_hh_b, W_out, b_out):
    i32 = jnp.int32
    word_idx = paths[..., 0].reshape(N, T).T.astype(i32)
    node_idx = nodes.reshape(-1).astype(i32)
    idx_a = jnp.concatenate(
        [word_idx[jnp.array(TS_A)].reshape(-1), node_idx,
         jnp.zeros((NPAD_H - 8 * N - 2 * B,), i32)])
    idx_b = word_idx[jnp.array(TS_B)].reshape(-1)

    table = _pad_cast(name_emb.T)
    rows_a = _sc_gather(table, idx_a, NPAD_H, CHUNK)
    rows_b = _sc_gather(table, idx_b, 8 * N, 32)

    pos_i = paths[..., 1].reshape(N, T).astype(i32)
    dep_i = paths[..., 2].reshape(N, T).astype(i32) + 40
    dir_i = paths[..., 3].reshape(N, T).astype(i32) + 90
    idxs = jnp.stack([pos_i, dep_i, dir_i])
    lens = edgecounts.reshape(N, 1).astype(i32)
    counts3 = counts.astype(jnp.float32).reshape(B, P, 1)

    sblk = jnp.zeros((SMALL, SDIM), jnp.float32)
    sblk = sblk.at[0:40, 0:4].set(pos_emb)
    sblk = sblk.at[40:90, 4:10].set(dep_emb)
    sblk = sblk.at[90:94, 10:13].set(dir_emb)

    def split_ih(w_ih):
        wp = _pad_gate_rows(w_ih)
        ww = jnp.pad(wp[:, :D].T, ((0, DG - D), (0, 0)))
        wsm = jnp.pad(wp[:, D:].T, ((0, SDIM - 13), (0, 0)))
        return ww.astype(jnp.bfloat16), wsm

    ww_f, wsm_f = split_ih(W_ih_f)
    ww_b, wsm_b = split_ih(W_ih_b)
    whh_f = jnp.pad(_pad_gate_rows(W_hh_f),
                    ((0, 0), (0, HP - HIDDEN))).T.astype(jnp.bfloat16)
    whh_b = jnp.pad(_pad_gate_rows(W_hh_b),
                    ((0, 0), (0, HP - HIDDEN))).T.astype(jnp.bfloat16)
    bias_f = _pad_gate_rows((b_ih_f + b_hh_f)[:, None]).reshape(1, G)
    bias_b = _pad_gate_rows((b_ih_b + b_hh_b)[:, None]).reshape(1, G)
    wout_n = jnp.zeros((2 * DG, NUM_REL), jnp.float32)
    wout_n = wout_n.at[0:D].set(W_out[:, :D].T)
    wout_n = wout_n.at[DG:DG + D].set(W_out[:, D:2 * D].T)
    wout_f = jnp.pad(W_out[:, 2 * D::2].T, ((0, HP - HIDDEN), (0, 0)))
    wout_b = jnp.pad(W_out[:, 2 * D + 1::2].T, ((0, HP - HIDDEN), (0, 0)))
    bout = b_out.reshape(1, NUM_REL)

    hc = pl.pallas_call(
        _tc_body1,
        out_shape=jax.ShapeDtypeStruct((4, N, HP), jnp.float32),
    )(rows_a, idxs, lens, sblk, wsm_f, wsm_b, ww_f, ww_b,
      whh_f, whh_b, bias_f, bias_b)
    out = pl.pallas_call(
        _tc_body2,
        out_shape=jax.ShapeDtypeStruct((B, NUM_REL), jnp.float32),
    )(rows_a, rows_b, idxs, lens, counts3, hc, sblk, wsm_f, wsm_b,
      ww_f, ww_b, whh_f, whh_b, bias_f, bias_b, wout_n, wout_f, wout_b, bout)
    return out


# speedup vs baseline: 1.0458x; 1.0054x over previous
import functools

import jax
import jax.numpy as jnp
from jax import lax
from jax.experimental import pallas as pl
from jax.experimental.pallas import tpu as pltpu
from jax.experimental.pallas import tpu_sc as plsc

D = 100
DG = 128
HIDDEN = 250
HP = 256
G = 4 * HP
B = 64
P = 16
T = 16
N = B * P
NUM_REL = 12
SMALL = 96
SDIM = 16
NIDX = N * T + 2 * B
NPAD_H = 8448
CHUNK = 88
TS_A = (0, 1, 2, 3, 12, 13, 14, 15)
TS_B = (4, 5, 6, 7, 8, 9, 10, 11)


def _pad_cast_body(src_ref, dst_ref):
    x = src_ref[...]
    xt = jnp.transpose(x, (1, 0))
    z = jnp.zeros((xt.shape[0], DG - D), xt.dtype)
    dst_ref[...] = jnp.concatenate([xt, z], axis=1)


def _pad_cast(name_emb_t):
    v, rows = name_emb_t.shape[1], 32768
    return pl.pallas_call(
        _pad_cast_body,
        grid=(pl.cdiv(v, rows),),
        in_specs=[pl.BlockSpec((D, rows), lambda i: (0, i))],
        out_specs=pl.BlockSpec((rows, DG), lambda i: (i, 0)),
        out_shape=jax.ShapeDtypeStruct((v, DG), jnp.float32),
    )(name_emb_t)


def _sc_gather(name_emb, idx_all, npad, chunk):
    info = plsc.get_sparse_core_info()
    nc, ns = info.num_cores, info.num_subcores
    nw = nc * ns
    bpw = npad // nw
    nchunk = bpw // chunk
    mesh = plsc.VectorSubcoreMesh(core_axis_name="c", subcore_axis_name="s")

    @functools.partial(
        pl.kernel,
        mesh=mesh,
        out_type=jax.ShapeDtypeStruct((npad, DG), jnp.float32),
        scratch_types=[
            pltpu.VMEM((nchunk, chunk), jnp.int32),
            pltpu.VMEM((bpw, DG), jnp.float32),
            pltpu.SemaphoreType.DMA,
        ],
    )
    def k(table_hbm, idx_hbm, out_hbm, idx_v, rows_v, sem):
        wid = lax.axis_index("s") * nc + lax.axis_index("c")
        base = wid * bpw
        pltpu.sync_copy(idx_hbm.at[wid], idx_v)
        copies = []
        for j in range(nchunk):
            copies.append(
                pltpu.async_copy(
                    table_hbm.at[idx_v.at[j]],
                    rows_v.at[pl.ds(j * chunk, chunk)],
                    sem,
                )
            )
        for c in copies:
            c.wait()
        pltpu.sync_copy(rows_v, out_hbm.at[pl.ds(base, bpw)])

    return k(name_emb, idx_all.reshape(nw, nchunk, chunk))


def _lstm_helpers(idxs_ref, sblk_ref, wsm_f_ref, wsm_b_ref):
    f32 = jnp.float32
    bf16 = jnp.bfloat16
    dot = functools.partial(jnp.dot, preferred_element_type=f32)
    sg_f = dot(sblk_ref[...], wsm_f_ref[...]).astype(bf16)
    sg_b = dot(sblk_ref[...], wsm_b_ref[...]).astype(bf16)
    iota = lax.broadcasted_iota(jnp.int32, (N, SMALL), 1)

    def onehot(t):
        p_ = idxs_ref[0, :, t:t + 1]
        d_ = idxs_ref[1, :, t:t + 1]
        r_ = idxs_ref[2, :, t:t + 1]
        hit = (iota == p_) | (iota == d_) | (iota == r_)
        return hit.astype(bf16)

    def cell(gates, h, c, upd):
        ig = jax.nn.sigmoid(gates[:, 0:HP])
        fg = jax.nn.sigmoid(gates[:, HP:2 * HP])
        gg = jnp.tanh(gates[:, 2 * HP:3 * HP])
        og = jax.nn.sigmoid(gates[:, 3 * HP:4 * HP])
        c2 = fg * c + ig * gg
        h2 = og * jnp.tanh(c2)
        return jnp.where(upd, h2, h), jnp.where(upd, c2, c)

    return sg_f, sg_b, onehot, cell


def _word_at(rowsA_ref, rowsB_ref, idxs_ref, t):
    if t in TS_B:
        x = rowsB_ref[pl.ds(TS_B.index(t) * N, N), :]
    else:
        x = rowsA_ref[pl.ds(TS_A.index(t) * N, N), :]
    return x.astype(jnp.bfloat16)


def _tc_body1(rowsA_ref, idxs_ref, len_ref, sblk_ref, wsm_f_ref, wsm_b_ref,
              ww_f_ref, ww_b_ref, whh_f_ref, whh_b_ref, bias_f_ref,
              bias_b_ref, hc_ref):
    f32 = jnp.float32
    bf16 = jnp.bfloat16
    dot = functools.partial(jnp.dot, preferred_element_type=f32)
    sg_f, sg_b, onehot, cell = _lstm_helpers(
        idxs_ref, sblk_ref, wsm_f_ref, wsm_b_ref)
    lens = len_ref[...]
    bias_f = bias_f_ref[...]
    bias_b = bias_b_ref[...]
    whh_f = whh_f_ref[...]
    whh_b = whh_b_ref[...]
    h_f = jnp.zeros((N, HP), f32)
    c_f = jnp.zeros((N, HP), f32)
    h_b = jnp.zeros((N, HP), f32)
    c_b = jnp.zeros((N, HP), f32)
    for s in range(4):
        tb = T - 1 - s
        g_f = (dot(_word_at(rowsA_ref, None, idxs_ref, s), ww_f_ref[...])
               + dot(onehot(s), sg_f) + dot(h_f.astype(bf16), whh_f) + bias_f)
        h_f, c_f = cell(g_f, h_f, c_f, lens > s)
        g_b = (dot(_word_at(rowsA_ref, None, idxs_ref, tb), ww_b_ref[...])
               + dot(onehot(tb), sg_b) + dot(h_b.astype(bf16), whh_b) + bias_b)
        h_b, c_b = cell(g_b, h_b, c_b, lens > tb)
    hc_ref[0] = h_f
    hc_ref[1] = c_f
    hc_ref[2] = h_b
    hc_ref[3] = c_b


def _tc_body2(rowsA_ref, rowsB_ref, idxs_ref, len_ref, counts_ref,
              hc_ref, sblk_ref, wsm_f_ref, wsm_b_ref, ww_f_ref, ww_b_ref,
              whh_f_ref, whh_b_ref, bias_f_ref, bias_b_ref,
              wout_n_ref, wout_f_ref, wout_b_ref, bout_ref, out_ref):
    f32 = jnp.float32
    bf16 = jnp.bfloat16
    dot = functools.partial(jnp.dot, preferred_element_type=f32)
    sg_f, sg_b, onehot, cell = _lstm_helpers(
        idxs_ref, sblk_ref, wsm_f_ref, wsm_b_ref)
    lens = len_ref[...]
    bias_f = bias_f_ref[...]
    bias_b = bias_b_ref[...]
    whh_f = whh_f_ref[...]
    whh_b = whh_b_ref[...]
    h_f = hc_ref[0]
    c_f = hc_ref[1]
    h_b = hc_ref[2]
    c_b = hc_ref[3]
    for s in range(4, T):
        tb = T - 1 - s
        g_f = (dot(_word_at(rowsA_ref, rowsB_ref, idxs_ref, s), ww_f_ref[...])
               + dot(onehot(s), sg_f) + dot(h_f.astype(bf16), whh_f) + bias_f)
        h_f, c_f = cell(g_f, h_f, c_f, lens > s)
        g_b = (dot(_word_at(rowsA_ref, rowsB_ref, idxs_ref, tb), ww_b_ref[...])
               + dot(onehot(tb), sg_b) + dot(h_b.astype(bf16), whh_b) + bias_b)
        h_b, c_b = cell(g_b, h_b, c_b, lens > tb)

    counts3 = counts_ref[...]
    pw_f = jnp.sum(h_f.reshape(B, P, HP) * counts3, axis=1)
    pw_b = jnp.sum(h_b.reshape(B, P, HP) * counts3, axis=1)
    nodes = rowsA_ref[pl.ds(8 * N, 2 * B), :].reshape(B, 2 * DG)
    logits = (dot(nodes, wout_n_ref[...]) + dot(pw_f, wout_f_ref[...])
              + dot(pw_b, wout_b_ref[...]) + bout_ref[...])
    mx = jnp.max(logits, axis=-1, keepdims=True)
    s_ = logits - mx
    lse = jnp.log(jnp.sum(jnp.exp(s_), axis=-1, keepdims=True))
    out_ref[...] = s_ - lse


def _pad_gate_rows(w):
    w4 = w.reshape(4, HIDDEN, -1)
    w4 = jnp.pad(w4, ((0, 0), (0, HP - HIDDEN), (0, 0)))
    return w4.reshape(G, -1)


def kernel(nodes, paths, counts, edgecounts, max_paths, max_edges, name_emb,
           pos_emb, dep_emb, dir_emb, W_ih_f, W_hh_f, b_ih_f, b_hh_f,
           W_ih_b, W_hh_b, b_ih_b, b_hh_b, W_out, b_out):
    i32 = jnp.int32
    word_idx = paths[..., 0].reshape(N, T).T.astype(i32)
    node_idx = nodes.reshape(-1).astype(i32)
    idx_a = jnp.concatenate(
        [word_idx[jnp.array(TS_A)].reshape(-1), node_idx,
         jnp.zeros((NPAD_H - 8 * N - 2 * B,), i32)])
    idx_b = word_idx[jnp.array(TS_B)].reshape(-1)

    table = _pad_cast(name_emb.T)
    rows_a = _sc_gather(table, idx_a, NPAD_H, CHUNK)
    rows_b = _sc_gather(table, idx_b, 8 * N, 128)

    pos_i = paths[..., 1].reshape(N, T).astype(i32)
    dep_i = paths[..., 2].reshape(N, T).astype(i32) + 40
    dir_i = paths[..., 3].reshape(N, T).astype(i32) + 90
    idxs = jnp.stack([pos_i, dep_i, dir_i])
    lens = edgecounts.reshape(N, 1).astype(i32)
    counts3 = counts.astype(jnp.float32).reshape(B, P, 1)

    sblk = jnp.zeros((SMALL, SDIM), jnp.float32)
    sblk = sblk.at[0:40, 0:4].set(pos_emb)
    sblk = sblk.at[40:90, 4:10].set(dep_emb)
    sblk = sblk.at[90:94, 10:13].set(dir_emb)

    def split_ih(w_ih):
        wp = _pad_gate_rows(w_ih)
        ww = jnp.pad(wp[:, :D].T, ((0, DG - D), (0, 0)))
        wsm = jnp.pad(wp[:, D:].T, ((0, SDIM - 13), (0, 0)))
        return ww.astype(jnp.bfloat16), wsm

    ww_f, wsm_f = split_ih(W_ih_f)
    ww_b, wsm_b = split_ih(W_ih_b)
    whh_f = jnp.pad(_pad_gate_rows(W_hh_f),
                    ((0, 0), (0, HP - HIDDEN))).T.astype(jnp.bfloat16)
    whh_b = jnp.pad(_pad_gate_rows(W_hh_b),
                    ((0, 0), (0, HP - HIDDEN))).T.astype(jnp.bfloat16)
    bias_f = _pad_gate_rows((b_ih_f + b_hh_f)[:, None]).reshape(1, G)
    bias_b = _pad_gate_rows((b_ih_b + b_hh_b)[:, None]).reshape(1, G)
    wout_n = jnp.zeros((2 * DG, NUM_REL), jnp.float32)
    wout_n = wout_n.at[0:D].set(W_out[:, :D].T)
    wout_n = wout_n.at[DG:DG + D].set(W_out[:, D:2 * D].T)
    wout_f = jnp.pad(W_out[:, 2 * D::2].T, ((0, HP - HIDDEN), (0, 0)))
    wout_b = jnp.pad(W_out[:, 2 * D + 1::2].T, ((0, HP - HIDDEN), (0, 0)))
    bout = b_out.reshape(1, NUM_REL)

    hc = pl.pallas_call(
        _tc_body1,
        out_shape=jax.ShapeDtypeStruct((4, N, HP), jnp.float32),
    )(rows_a, idxs, lens, sblk, wsm_f, wsm_b, ww_f, ww_b,
      whh_f, whh_b, bias_f, bias_b)
    out = pl.pallas_call(
        _tc_body2,
        out_shape=jax.ShapeDtypeStruct((B, NUM_REL), jnp.float32),
    )(rows_a, rows_b, idxs, lens, counts3, hc, sblk, wsm_f, wsm_b,
      ww_f, ww_b, whh_f, whh_b, bias_f, bias_b, wout_n, wout_f, wout_b, bout)
    return out
